# baseline (speedup 1.0000x reference)
import functools

import jax
import jax.numpy as jnp
from jax import lax
from jax.experimental import pallas as pl
from jax.experimental.pallas import tpu as pltpu
from jax.experimental.pallas import tpu_sc as plsc

_INTERPRET = False

_B, _S, _D, _F, _E, _K = 2, 2048, 1024, 2048, 8, 2
_T = _B * _S
_TK = _T * _K
_BM = 256
_NPAD = _TK + _E * _BM
_NBLK = _NPAD // _BM
_BT = 512


def _block1_router_body(x_ref, n1w_ref, n1b_ref, mw_ref, mb_ref,
                        n2w_ref, n2b_ref, rw_ref,
                        xmid_ref, h2_ref, ek_ref, wk_ref):
    x = x_ref[...]
    m = jnp.mean(x, axis=1, keepdims=True)
    v = jnp.mean((x - m) * (x - m), axis=1, keepdims=True)
    h = (x - m) * lax.rsqrt(v + 1e-5) * n1w_ref[...][None, :] + n1b_ref[...][None, :]
    h = h + lax.dot_general(h, mw_ref[...], (((1,), (1,)), ((), ())),
                            preferred_element_type=jnp.float32) + mb_ref[...][None, :]
    xm = x + h
    xmid_ref[...] = xm
    m2 = jnp.mean(xm, axis=1, keepdims=True)
    v2 = jnp.mean((xm - m2) * (xm - m2), axis=1, keepdims=True)
    h2 = (xm - m2) * lax.rsqrt(v2 + 1e-5) * n2w_ref[...][None, :] + n2b_ref[...][None, :]
    h2_ref[...] = h2
    logits = lax.dot_general(h2, rw_ref[...], (((1,), (1,)), ((), ())),
                             preferred_element_type=jnp.float32)
    z = logits - jnp.max(logits, axis=1, keepdims=True)
    ez = jnp.exp(z)
    probs = ez / jnp.sum(ez, axis=1, keepdims=True)
    eiota = lax.broadcasted_iota(jnp.int32, probs.shape, 1)
    w0 = jnp.max(probs, axis=1, keepdims=True)
    i0 = jnp.min(jnp.where(probs == w0, eiota, _E), axis=1, keepdims=True)
    p2 = jnp.where(eiota == i0, -1.0, probs)
    w1 = jnp.max(p2, axis=1, keepdims=True)
    i1 = jnp.min(jnp.where(p2 == w1, eiota, _E), axis=1, keepdims=True)
    ek_ref[...] = jnp.concatenate([i0, i1], axis=1)
    wk_ref[...] = jnp.concatenate([w0, w1], axis=1)


def _stage_a(x2d, n1w, n1b, mw, mb, n2w, n2b, rw):
    grid = (_T // _BT,)
    return pl.pallas_call(
        _block1_router_body,
        grid=grid,
        in_specs=[
            pl.BlockSpec((_BT, _D), lambda i: (i, 0)),
            pl.BlockSpec((_D,), lambda i: (0,)),
            pl.BlockSpec((_D,), lambda i: (0,)),
            pl.BlockSpec((_D, _D), lambda i: (0, 0)),
            pl.BlockSpec((_D,), lambda i: (0,)),
            pl.BlockSpec((_D,), lambda i: (0,)),
            pl.BlockSpec((_D,), lambda i: (0,)),
            pl.BlockSpec((_E, _D), lambda i: (0, 0)),
        ],
        out_specs=[
            pl.BlockSpec((_BT, _D), lambda i: (i, 0)),
            pl.BlockSpec((_BT, _D), lambda i: (i, 0)),
            pl.BlockSpec((_BT, _K), lambda i: (i, 0)),
            pl.BlockSpec((_BT, _K), lambda i: (i, 0)),
        ],
        out_shape=[
            jax.ShapeDtypeStruct((_T, _D), jnp.float32),
            jax.ShapeDtypeStruct((_T, _D), jnp.float32),
            jax.ShapeDtypeStruct((_T, _K), jnp.int32),
            jax.ShapeDtypeStruct((_T, _K), jnp.float32),
        ],
        interpret=_INTERPRET,
    )(x2d, n1w, n1b, mw, mb, n2w, n2b, rw)


def _dispatch_meta(e_flat, w_flat):
    eids = jnp.arange(_E, dtype=jnp.int32)
    onehot = (e_flat[None, :] == eids[:, None]).astype(jnp.int32)
    within = jnp.cumsum(onehot, axis=1)
    rank = jnp.take_along_axis(within, e_flat[None, :], axis=0)[0] - 1
    counts = within[:, -1]
    padded = ((counts + _BM - 1) // _BM) * _BM
    gstart = jnp.concatenate([jnp.zeros((1,), jnp.int32),
                              jnp.cumsum(padded)[:-1].astype(jnp.int32)])
    dst = gstart[e_flat] + rank
    tok = jnp.arange(_TK, dtype=jnp.int32) // _K
    pad_init = jnp.arange(_NPAD, dtype=jnp.int32) % _T
    tok_sorted = pad_init.at[dst].set(tok)
    w_sorted = jnp.zeros((_NPAD,), jnp.float32).at[dst].set(w_flat)
    gend = (gstart + padded).astype(jnp.int32)
    blkid = jnp.arange(_NBLK, dtype=jnp.int32)
    blk_expert = jnp.minimum(
        jnp.sum((blkid[:, None] * _BM >= gend[None, :]).astype(jnp.int32), axis=1),
        _E - 1).astype(jnp.int32)
    return tok_sorted, w_sorted, blk_expert, dst


def _sc_gather_rows(table, idx, nch):
    n, d = idx.shape[0], table.shape[1]
    info = plsc.get_sparse_core_info()
    nw = info.num_cores * info.num_subcores
    per_w = n // nw
    chunk = per_w // nch
    mesh = plsc.VectorSubcoreMesh(core_axis_name="c", subcore_axis_name="s")

    @functools.partial(
        pl.kernel, mesh=mesh,
        out_type=jax.ShapeDtypeStruct((n, d), jnp.float32),
        scratch_types=[
            pltpu.VMEM((per_w,), jnp.int32),
            pltpu.VMEM((chunk, d), jnp.float32),
            pltpu.VMEM((chunk, d), jnp.float32),
            pltpu.SemaphoreType.DMA,
            pltpu.SemaphoreType.DMA,
            pltpu.SemaphoreType.DMA,
            pltpu.SemaphoreType.DMA,
        ],
    )
    def k(table_hbm, idx_hbm, out_hbm, idx_v, buf0, buf1, g0, g1, w0, w1):
        wid = lax.axis_index("s") * info.num_cores + lax.axis_index("c")
        base = wid * per_w
        pltpu.sync_copy(idx_hbm.at[pl.ds(base, per_w)], idx_v)
        bufs, gsem, wsem = (buf0, buf1), (g0, g1), (w0, w1)
        gath, wr = {}, {}
        for c in range(nch):
            b = c % 2
            if c >= 2:
                wr[c - 2].wait()
            gath[c] = pltpu.async_copy(
                table_hbm.at[idx_v.at[pl.ds(c * chunk, chunk)]], bufs[b], gsem[b])
            if c >= 1:
                pb = (c - 1) % 2
                gath[c - 1].wait()
                wr[c - 1] = pltpu.async_copy(
                    bufs[pb], out_hbm.at[pl.ds(base + (c - 1) * chunk, chunk)],
                    wsem[pb])
        gath[nch - 1].wait()
        wr[nch - 1] = pltpu.async_copy(
            bufs[(nch - 1) % 2],
            out_hbm.at[pl.ds(base + (nch - 1) * chunk, chunk)],
            wsem[(nch - 1) % 2])
        if nch >= 2:
            wr[nch - 2].wait()
        wr[nch - 1].wait()

    return k(table, idx)


def _ffn_body(s_ref, x_ref, w1_ref, b1_ref, w2_ref, b2_ref, ws_ref, y_ref):
    xb = x_ref[...].astype(jnp.bfloat16)
    h = lax.dot_general(xb, w1_ref[0], (((1,), (0,)), ((), ())),
                        preferred_element_type=jnp.float32) + b1_ref[0]
    h = h * (1.0 / (1.0 + jnp.exp(-h)))
    y = lax.dot_general(h.astype(jnp.bfloat16), w2_ref[0], (((1,), (0,)), ((), ())),
                        preferred_element_type=jnp.float32) + b2_ref[0]
    y_ref[...] = y * ws_ref[...][:, None]


def _stage_c(blk_expert, xs, w1, b1, w2, b2, ws):
    grid_spec = pltpu.PrefetchScalarGridSpec(
        num_scalar_prefetch=1,
        grid=(_NBLK,),
        in_specs=[
            pl.BlockSpec((_BM, _D), lambda i, s: (i, 0)),
            pl.BlockSpec((1, _D, _F), lambda i, s: (s[i], 0, 0)),
            pl.BlockSpec((1, 1, _F), lambda i, s: (s[i], 0, 0)),
            pl.BlockSpec((1, _F, _D), lambda i, s: (s[i], 0, 0)),
            pl.BlockSpec((1, 1, _D), lambda i, s: (s[i], 0, 0)),
            pl.BlockSpec((_BM,), lambda i, s: (i,)),
        ],
        out_specs=pl.BlockSpec((_BM, _D), lambda i, s: (i, 0)),
    )
    return pl.pallas_call(
        _ffn_body,
        grid_spec=grid_spec,
        out_shape=jax.ShapeDtypeStruct((_NPAD, _D), jnp.float32),
        interpret=_INTERPRET,
    )(blk_expert, xs, w1.astype(jnp.bfloat16), b1.reshape(_E, 1, _F),
      w2.astype(jnp.bfloat16), b2.reshape(_E, 1, _D), ws)


def _combine_body(xmid_ref, yp_ref, out_ref):
    out_ref[...] = xmid_ref[...] + yp_ref[:, 0, :] + yp_ref[:, 1, :]


def _stage_e(xmid, ypair3):
    grid = (_T // _BT,)
    return pl.pallas_call(
        _combine_body,
        grid=grid,
        in_specs=[
            pl.BlockSpec((_BT, _D), lambda i: (i, 0)),
            pl.BlockSpec((_BT, _K, _D), lambda i: (i, 0, 0)),
        ],
        out_specs=pl.BlockSpec((_BT, _D), lambda i: (i, 0)),
        out_shape=jax.ShapeDtypeStruct((_T, _D), jnp.float32),
        interpret=_INTERPRET,
    )(xmid, ypair3)


def kernel(x, norm1_w, norm1_b, mixer_W, mixer_b, norm2_w, norm2_b,
           router_W, W1, b1, W2, b2):
    x2d = x.reshape(_T, _D)
    xmid, h2, ek, wk = _stage_a(x2d, norm1_w, norm1_b, mixer_W, mixer_b,
                                norm2_w, norm2_b, router_W)
    e_flat = ek.reshape(_TK)
    w_flat = wk.reshape(_TK)
    tok_sorted, w_sorted, blk_expert, dst = _dispatch_meta(e_flat, w_flat)
    xs = _sc_gather_rows(h2, tok_sorted, nch=8)
    y = _stage_c(blk_expert, xs, W1, b1, W2, b2, w_sorted)
    ypair = _sc_gather_rows(y, dst, nch=8)
    out = _stage_e(xmid, ypair.reshape(_T, _K, _D))
    return out.reshape(_B, _S, _D)


# speedup vs baseline: 1.2359x; 1.2359x over previous
import functools

import jax
import jax.numpy as jnp
from jax import lax
from jax.experimental import pallas as pl
from jax.experimental.pallas import tpu as pltpu
from jax.experimental.pallas import tpu_sc as plsc

_INTERPRET = False

_B, _S, _D, _F, _E, _K = 2, 2048, 1024, 2048, 8, 2
_T = _B * _S
_TK = _T * _K
_BM = 256
_NPAD = _TK + _E * _BM
_NBLK = _NPAD // _BM
_BT = 512


def _block1_router_body(x_ref, n1w_ref, n1b_ref, mw_ref, mb_ref,
                        n2w_ref, n2b_ref, rw_ref,
                        xmid_ref, h2_ref, ek_ref, wk_ref):
    x = x_ref[...]
    m = jnp.mean(x, axis=1, keepdims=True)
    v = jnp.mean((x - m) * (x - m), axis=1, keepdims=True)
    h = (x - m) * lax.rsqrt(v + 1e-5) * n1w_ref[...][None, :] + n1b_ref[...][None, :]
    h = h + lax.dot_general(h, mw_ref[...], (((1,), (1,)), ((), ())),
                            preferred_element_type=jnp.float32) + mb_ref[...][None, :]
    xm = x + h
    xmid_ref[...] = xm
    m2 = jnp.mean(xm, axis=1, keepdims=True)
    v2 = jnp.mean((xm - m2) * (xm - m2), axis=1, keepdims=True)
    h2 = (xm - m2) * lax.rsqrt(v2 + 1e-5) * n2w_ref[...][None, :] + n2b_ref[...][None, :]
    h2_ref[...] = h2
    logits = lax.dot_general(h2, rw_ref[...], (((1,), (1,)), ((), ())),
                             preferred_element_type=jnp.float32)
    z = logits - jnp.max(logits, axis=1, keepdims=True)
    ez = jnp.exp(z)
    probs = ez / jnp.sum(ez, axis=1, keepdims=True)
    eiota = lax.broadcasted_iota(jnp.int32, probs.shape, 1)
    w0 = jnp.max(probs, axis=1, keepdims=True)
    i0 = jnp.min(jnp.where(probs == w0, eiota, _E), axis=1, keepdims=True)
    p2 = jnp.where(eiota == i0, -1.0, probs)
    w1 = jnp.max(p2, axis=1, keepdims=True)
    i1 = jnp.min(jnp.where(p2 == w1, eiota, _E), axis=1, keepdims=True)
    ek_ref[...] = jnp.concatenate([i0, i1], axis=1)
    wk_ref[...] = jnp.concatenate([w0, w1], axis=1)


def _stage_a(x2d, n1w, n1b, mw, mb, n2w, n2b, rw):
    grid = (_T // _BT,)
    return pl.pallas_call(
        _block1_router_body,
        grid=grid,
        in_specs=[
            pl.BlockSpec((_BT, _D), lambda i: (i, 0)),
            pl.BlockSpec((_D,), lambda i: (0,)),
            pl.BlockSpec((_D,), lambda i: (0,)),
            pl.BlockSpec((_D, _D), lambda i: (0, 0)),
            pl.BlockSpec((_D,), lambda i: (0,)),
            pl.BlockSpec((_D,), lambda i: (0,)),
            pl.BlockSpec((_D,), lambda i: (0,)),
            pl.BlockSpec((_E, _D), lambda i: (0, 0)),
        ],
        out_specs=[
            pl.BlockSpec((_BT, _D), lambda i: (i, 0)),
            pl.BlockSpec((_BT, _D), lambda i: (i, 0)),
            pl.BlockSpec((_BT, _K), lambda i: (i, 0)),
            pl.BlockSpec((_BT, _K), lambda i: (i, 0)),
        ],
        out_shape=[
            jax.ShapeDtypeStruct((_T, _D), jnp.float32),
            jax.ShapeDtypeStruct((_T, _D), jnp.float32),
            jax.ShapeDtypeStruct((_T, _K), jnp.int32),
            jax.ShapeDtypeStruct((_T, _K), jnp.float32),
        ],
        interpret=_INTERPRET,
    )(x2d, n1w, n1b, mw, mb, n2w, n2b, rw)


def _dispatch_meta(e_flat, w_flat):
    eids = jnp.arange(_E, dtype=jnp.int32)
    onehot = (e_flat[None, :] == eids[:, None]).astype(jnp.int32)
    within = jnp.cumsum(onehot, axis=1)
    rank = jnp.sum(within * onehot, axis=0) - 1
    counts = within[:, -1]
    padded = ((counts + _BM - 1) // _BM) * _BM
    gstart = jnp.concatenate([jnp.zeros((1,), jnp.int32),
                              jnp.cumsum(padded)[:-1].astype(jnp.int32)])
    dst = jnp.sum(onehot * gstart[:, None], axis=0) + rank
    tok = jnp.arange(_TK, dtype=jnp.int32) // _K
    pad_init = jnp.arange(_NPAD, dtype=jnp.int32) % _T
    tok_sorted = pad_init.at[dst].set(tok)
    w_sorted = jnp.zeros((_NPAD,), jnp.float32).at[dst].set(w_flat)
    gend = (gstart + padded).astype(jnp.int32)
    blkid = jnp.arange(_NBLK, dtype=jnp.int32)
    blk_expert = jnp.minimum(
        jnp.sum((blkid[:, None] * _BM >= gend[None, :]).astype(jnp.int32), axis=1),
        _E - 1).astype(jnp.int32)
    return tok_sorted, w_sorted, blk_expert, dst


def _sc_gather_rows(table, idx, nch, n=None):
    n, d = n if n is not None else idx.shape[0], table.shape[1]
    info = plsc.get_sparse_core_info()
    nw = info.num_cores * info.num_subcores
    per_w = n // nw
    chunk = per_w // nch
    mesh = plsc.VectorSubcoreMesh(core_axis_name="c", subcore_axis_name="s")

    @functools.partial(
        pl.kernel, mesh=mesh,
        out_type=jax.ShapeDtypeStruct((n, d), jnp.float32),
        scratch_types=[
            pltpu.VMEM((per_w,), jnp.int32),
            pltpu.VMEM((chunk, d), jnp.float32),
            pltpu.VMEM((chunk, d), jnp.float32),
            pltpu.SemaphoreType.DMA,
            pltpu.SemaphoreType.DMA,
            pltpu.SemaphoreType.DMA,
            pltpu.SemaphoreType.DMA,
        ],
    )
    def k(table_hbm, idx_hbm, out_hbm, idx_v, buf0, buf1, g0, g1, w0, w1):
        wid = lax.axis_index("s") * info.num_cores + lax.axis_index("c")
        base = wid * per_w
        pltpu.sync_copy(idx_hbm.at[pl.ds(base, per_w)], idx_v)
        bufs, gsem, wsem = (buf0, buf1), (g0, g1), (w0, w1)
        gath, wr = {}, {}
        for c in range(nch):
            b = c % 2
            if c >= 2:
                wr[c - 2].wait()
            gath[c] = pltpu.async_copy(
                table_hbm.at[idx_v.at[pl.ds(c * chunk, chunk)]], bufs[b], gsem[b])
            if c >= 1:
                pb = (c - 1) % 2
                gath[c - 1].wait()
                wr[c - 1] = pltpu.async_copy(
                    bufs[pb], out_hbm.at[pl.ds(base + (c - 1) * chunk, chunk)],
                    wsem[pb])
        gath[nch - 1].wait()
        wr[nch - 1] = pltpu.async_copy(
            bufs[(nch - 1) % 2],
            out_hbm.at[pl.ds(base + (nch - 1) * chunk, chunk)],
            wsem[(nch - 1) % 2])
        if nch >= 2:
            wr[nch - 2].wait()
        wr[nch - 1].wait()

    return k(table, idx)


def _ffn_body(s_ref, x_ref, w1_ref, b1_ref, w2_ref, b2_ref, ws_ref, y_ref):
    xb = x_ref[...]
    h = lax.dot_general(xb, w1_ref[0], (((1,), (0,)), ((), ())),
                        preferred_element_type=jnp.float32) + b1_ref[0]
    h = h * (1.0 / (1.0 + jnp.exp(-h)))
    y = lax.dot_general(h, w2_ref[0], (((1,), (0,)), ((), ())),
                        preferred_element_type=jnp.float32) + b2_ref[0]
    y_ref[...] = y * ws_ref[...][:, None]


def _stage_c(blk_expert, xs, w1, b1, w2, b2, ws):
    grid_spec = pltpu.PrefetchScalarGridSpec(
        num_scalar_prefetch=1,
        grid=(_NBLK,),
        in_specs=[
            pl.BlockSpec((_BM, _D), lambda i, s: (i, 0)),
            pl.BlockSpec((1, _D, _F), lambda i, s: (s[i], 0, 0)),
            pl.BlockSpec((1, 1, _F), lambda i, s: (s[i], 0, 0)),
            pl.BlockSpec((1, _F, _D), lambda i, s: (s[i], 0, 0)),
            pl.BlockSpec((1, 1, _D), lambda i, s: (s[i], 0, 0)),
            pl.BlockSpec((_BM,), lambda i, s: (i,)),
        ],
        out_specs=pl.BlockSpec((_BM, _D), lambda i, s: (i, 0)),
    )
    return pl.pallas_call(
        _ffn_body,
        grid_spec=grid_spec,
        out_shape=jax.ShapeDtypeStruct((_NPAD, _D), jnp.float32),
        interpret=_INTERPRET,
    )(blk_expert, xs, w1, b1.reshape(_E, 1, _F), w2, b2.reshape(_E, 1, _D), ws)


def _combine_body(xmid_ref, yp_ref, out_ref):
    out_ref[...] = xmid_ref[...] + yp_ref[:, 0, :] + yp_ref[:, 1, :]


def _stage_e(xmid, ypair3):
    grid = (_T // _BT,)
    return pl.pallas_call(
        _combine_body,
        grid=grid,
        in_specs=[
            pl.BlockSpec((_BT, _D), lambda i: (i, 0)),
            pl.BlockSpec((_BT, _K, _D), lambda i: (i, 0, 0)),
        ],
        out_specs=pl.BlockSpec((_BT, _D), lambda i: (i, 0)),
        out_shape=jax.ShapeDtypeStruct((_T, _D), jnp.float32),
        interpret=_INTERPRET,
    )(xmid, ypair3)


def kernel(x, norm1_w, norm1_b, mixer_W, mixer_b, norm2_w, norm2_b,
           router_W, W1, b1, W2, b2):
    x2d = x.reshape(_T, _D)
    xmid, h2, ek, wk = _stage_a(x2d, norm1_w, norm1_b, mixer_W, mixer_b,
                                norm2_w, norm2_b, router_W)
    tok_sorted, w_sorted, blk_expert, dst = _dispatch_meta(
        ek.reshape(_TK), wk.reshape(_TK))
    xs = _sc_gather_rows(h2, tok_sorted, nch=8)
    y = _stage_c(blk_expert, xs, W1, b1, W2, b2, w_sorted)
    ypair = _sc_gather_rows(y, dst, nch=8)
    out = _stage_e(xmid, ypair.reshape(_T, _K, _D))
    return out.reshape(_B, _S, _D)
